# baseline (speedup 1.0000x reference)
import jax
import jax.numpy as jnp
from jax.experimental import pallas as pl
from jax.experimental.pallas import tpu as pltpu

TOKENS = 8192
IN_CHANNELS = 2048
NUM_EXPERTS = 64
TILE_M = 1024


def _gate_softmax_kernel(x_ref, wt_ref, b_ref, o_ref):
    xb = x_ref[...].astype(jnp.bfloat16)
    logits = jnp.dot(xb, wt_ref[...],
                     preferred_element_type=jnp.float32) + b_ref[...]
    m = jnp.max(logits, axis=1, keepdims=True)
    e = jnp.exp(logits - m)
    o_ref[...] = e / jnp.sum(e, axis=1, keepdims=True)


def kernel(x, W, b):
    wt = W.T.astype(jnp.bfloat16)
    b2 = b.reshape(1, NUM_EXPERTS)
    grid = (TOKENS // TILE_M,)
    return pl.pallas_call(
        _gate_softmax_kernel,
        grid=grid,
        in_specs=[
            pl.BlockSpec((TILE_M, IN_CHANNELS), lambda i: (i, 0)),
            pl.BlockSpec((IN_CHANNELS, NUM_EXPERTS), lambda i: (0, 0)),
            pl.BlockSpec((1, NUM_EXPERTS), lambda i: (0, 0)),
        ],
        out_specs=pl.BlockSpec((TILE_M, NUM_EXPERTS), lambda i: (i, 0)),
        out_shape=jax.ShapeDtypeStruct((TOKENS, NUM_EXPERTS), jnp.float32),
        compiler_params=pltpu.CompilerParams(
            dimension_semantics=("arbitrary",),
        ),
    )(x, wt, b2)


# speedup vs baseline: 1.0762x; 1.0762x over previous
import jax
import jax.numpy as jnp
from jax.experimental import pallas as pl
from jax.experimental.pallas import tpu as pltpu

TOKENS = 8192
IN_CHANNELS = 2048
NUM_EXPERTS = 64
TILE_M = 1024


def _gate_softmax_kernel(x_ref, w_ref, b_ref, o_ref):
    logits = jax.lax.dot_general(
        x_ref[...], w_ref[...], (((1,), (1,)), ((), ())),
        preferred_element_type=jnp.float32) + b_ref[...]
    m = jnp.max(logits, axis=1, keepdims=True)
    e = jnp.exp(logits - m)
    o_ref[...] = e / jnp.sum(e, axis=1, keepdims=True)


def kernel(x, W, b):
    b2 = b.reshape(1, NUM_EXPERTS)
    grid = (TOKENS // TILE_M,)
    return pl.pallas_call(
        _gate_softmax_kernel,
        grid=grid,
        in_specs=[
            pl.BlockSpec((TILE_M, IN_CHANNELS), lambda i: (i, 0)),
            pl.BlockSpec((NUM_EXPERTS, IN_CHANNELS), lambda i: (0, 0)),
            pl.BlockSpec((1, NUM_EXPERTS), lambda i: (0, 0)),
        ],
        out_specs=pl.BlockSpec((TILE_M, NUM_EXPERTS), lambda i: (i, 0)),
        out_shape=jax.ShapeDtypeStruct((TOKENS, NUM_EXPERTS), jnp.float32),
        compiler_params=pltpu.CompilerParams(
            dimension_semantics=("arbitrary",),
        ),
    )(x, W, b2)


# speedup vs baseline: 1.0807x; 1.0042x over previous
import jax
import jax.numpy as jnp
from jax.experimental import pallas as pl
from jax.experimental.pallas import tpu as pltpu

TOKENS = 8192
IN_CHANNELS = 2048
NUM_EXPERTS = 64
TILE_M = 1024


def _gate_softmax_kernel(x_ref, w_ref, b_ref, o_ref):
    logits = jax.lax.dot_general(
        x_ref[...], w_ref[...], (((1,), (1,)), ((), ())),
        preferred_element_type=jnp.float32) + b_ref[...][None, :]
    m = jnp.max(logits, axis=1, keepdims=True)
    e = jnp.exp(logits - m)
    o_ref[...] = e / jnp.sum(e, axis=1, keepdims=True)


def kernel(x, W, b):
    grid = (TOKENS // TILE_M,)
    return pl.pallas_call(
        _gate_softmax_kernel,
        grid=grid,
        in_specs=[
            pl.BlockSpec((TILE_M, IN_CHANNELS), lambda i: (i, 0)),
            pl.BlockSpec((NUM_EXPERTS, IN_CHANNELS), lambda i: (0, 0)),
            pl.BlockSpec((NUM_EXPERTS,), lambda i: (0,)),
        ],
        out_specs=pl.BlockSpec((TILE_M, NUM_EXPERTS), lambda i: (i, 0)),
        out_shape=jax.ShapeDtypeStruct((TOKENS, NUM_EXPERTS), jnp.float32),
        compiler_params=pltpu.CompilerParams(
            dimension_semantics=("arbitrary",),
        ),
    )(x, W, b)
